# baseline (speedup 1.0000x reference)
import functools

import jax
import jax.numpy as jnp
from jax import lax
from jax.experimental import pallas as pl
from jax.experimental.pallas import tpu as pltpu
from jax.experimental.pallas import tpu_sc as plsc

NUM_EXPERTS = 16
TOP_K = 2
N_TOKENS = 4096
D_MODEL = 1024
N_PAIRS = N_TOKENS * TOP_K
BB = 256
P_MAX = N_PAIRS + NUM_EXPERTS * BB
NB = P_MAX // BB
NBLK_PAD = 64
NC, NS, L = 2, 16, 16
BT = 1024
NBT = N_TOKENS // BT
TW = N_TOKENS // (NC * NS)
TCH = 16
NCH = TW // TCH


def _router_body(x_ref, wr_ref, br_ref, ei_ref, g0_ref, g1_ref, hcnt_ref, xb_ref):
    logits = (
        jnp.dot(x_ref[...], wr_ref[...], preferred_element_type=jnp.float32)
        + br_ref[...]
    )
    iota = jax.lax.broadcasted_iota(jnp.int32, logits.shape, 1)
    m1 = jnp.max(logits, axis=-1, keepdims=True)
    e0 = jnp.min(
        jnp.where(logits >= m1, iota, NUM_EXPERTS), axis=-1, keepdims=True
    )
    masked = jnp.where(logits >= m1, -jnp.inf, logits)
    m2 = jnp.max(masked, axis=-1, keepdims=True)
    e1 = jnp.min(
        jnp.where(masked >= m2, iota, NUM_EXPERTS), axis=-1, keepdims=True
    )
    t = jnp.exp(m2 - m1)
    g0 = 1.0 / (1.0 + t)
    ei_ref[...] = jnp.concatenate([e0, e1], axis=1)
    g0_ref[...] = g0.reshape(BT // L, L)
    g1_ref[...] = (1.0 - g0).reshape(BT // L, L)
    h = (iota == e0).astype(jnp.float32) + (iota == e1).astype(jnp.float32)
    hcnt_ref[...] = jnp.sum(h, axis=0, keepdims=True).reshape(1, 1, NUM_EXPERTS)
    xb_ref[...] = x_ref[...].astype(jnp.bfloat16)


def _router(x, W_router, b_router):
    return pl.pallas_call(
        _router_body,
        grid=(NBT,),
        in_specs=[
            pl.BlockSpec((BT, D_MODEL), lambda i: (i, 0)),
            pl.BlockSpec((D_MODEL, NUM_EXPERTS), lambda i: (0, 0)),
            pl.BlockSpec((1, NUM_EXPERTS), lambda i: (0, 0)),
        ],
        out_specs=[
            pl.BlockSpec((BT, TOP_K), lambda i: (i, 0)),
            pl.BlockSpec((BT // L, L), lambda i: (i, 0)),
            pl.BlockSpec((BT // L, L), lambda i: (i, 0)),
            pl.BlockSpec((1, 1, NUM_EXPERTS), lambda i: (i, 0, 0)),
            pl.BlockSpec((BT, D_MODEL), lambda i: (i, 0)),
        ],
        out_shape=[
            jax.ShapeDtypeStruct((N_TOKENS, TOP_K), jnp.int32),
            jax.ShapeDtypeStruct((N_TOKENS // L, L), jnp.float32),
            jax.ShapeDtypeStruct((N_TOKENS // L, L), jnp.float32),
            jax.ShapeDtypeStruct((NBT, 1, NUM_EXPERTS), jnp.float32),
            jax.ShapeDtypeStruct((N_TOKENS, D_MODEL), jnp.bfloat16),
        ],
    )(x, W_router, b_router.reshape(1, NUM_EXPERTS))


def _plan_body(ei_ref, hcnt_ref, pos0_ref, pos1_ref, blk_ref, carry_scr):
    i = pl.program_id(0)

    @pl.when(i == 0)
    def _():
        carry_scr[...] = jnp.zeros_like(carry_scr)

    e0 = ei_ref[:, 0:1]
    e1 = ei_ref[:, 1:2]
    io = jax.lax.broadcasted_iota(jnp.int32, (BT, NUM_EXPERTS), 1)
    oh0 = (io == e0).astype(jnp.float32)
    oh1 = (io == e1).astype(jnp.float32)
    h = oh0 + oh1

    counts = jnp.sum(hcnt_ref[...], axis=0)
    pe = jnp.ceil(counts * (1.0 / BB)) * BB
    r16 = jax.lax.broadcasted_iota(jnp.int32, (NUM_EXPERTS, NUM_EXPERTS), 0)
    c16 = jax.lax.broadcasted_iota(jnp.int32, (NUM_EXPERTS, NUM_EXPERTS), 1)
    tri16 = (r16 <= c16).astype(jnp.float32)
    ends = jnp.dot(pe, tri16, preferred_element_type=jnp.float32)
    base = ends - pe

    rr = jax.lax.broadcasted_iota(jnp.int32, (BT, BT), 0)
    cc = jax.lax.broadcasted_iota(jnp.int32, (BT, BT), 1)
    tstrict = (cc < rr).astype(jnp.float32)
    ex = jnp.dot(tstrict, h, preferred_element_type=jnp.float32) + carry_scr[...]

    rank0 = jnp.sum(ex * oh0, axis=1, keepdims=True)
    rank1 = jnp.sum(ex * oh1, axis=1, keepdims=True)
    base0 = jnp.sum(base * oh0, axis=1, keepdims=True)
    base1 = jnp.sum(base * oh1, axis=1, keepdims=True)
    p0 = (base0 + rank0).astype(jnp.int32)
    p1 = (base1 + rank1).astype(jnp.int32)
    pos0_ref[...] = p0.reshape(BT // L, L)
    pos1_ref[...] = p1.reshape(BT // L, L)

    carry_scr[...] += jnp.sum(h, axis=0, keepdims=True)

    @pl.when(i == NBT - 1)
    def _():
        bstart = (
            jax.lax.broadcasted_iota(jnp.int32, (NBLK_PAD, NUM_EXPERTS), 0) * BB
        ).astype(jnp.float32)
        acc = jnp.sum((bstart >= ends).astype(jnp.float32), axis=1, keepdims=True)
        blk_ref[...] = jnp.minimum(acc, NUM_EXPERTS - 1).astype(jnp.int32)


def _plan(ei, hcnt):
    return pl.pallas_call(
        _plan_body,
        grid=(NBT,),
        in_specs=[
            pl.BlockSpec((BT, TOP_K), lambda i: (i, 0)),
            pl.BlockSpec((NBT, 1, NUM_EXPERTS), lambda i: (0, 0, 0)),
        ],
        out_specs=[
            pl.BlockSpec((BT // L, L), lambda i: (i, 0)),
            pl.BlockSpec((BT // L, L), lambda i: (i, 0)),
            pl.BlockSpec((NBLK_PAD, 1), lambda i: (0, 0)),
        ],
        out_shape=[
            jax.ShapeDtypeStruct((N_TOKENS // L, L), jnp.int32),
            jax.ShapeDtypeStruct((N_TOKENS // L, L), jnp.int32),
            jax.ShapeDtypeStruct((NBLK_PAD, 1), jnp.int32),
        ],
        scratch_shapes=[pltpu.VMEM((1, NUM_EXPERTS), jnp.float32)],
        compiler_params=pltpu.CompilerParams(
            dimension_semantics=("arbitrary",),
        ),
    )(ei, hcnt)


def _scatter_body(
    x_hbm, pos0_hbm, pos1_hbm, xg_hbm,
    p0v, p1v, xb0, xb1, semA0, semB0, semA1, semB1,
):
    c = lax.axis_index("c")
    s = lax.axis_index("s")
    wid = s * NC + c
    row0 = pl.multiple_of(wid * (TW // TCH), 8)
    pltpu.sync_copy(pos0_hbm.at[pl.ds(row0, NCH)], p0v)
    pltpu.sync_copy(pos1_hbm.at[pl.ds(row0, NCH)], p1v)

    xbufs = (xb0, xb1)
    semsA = (semA0, semA1)
    semsB = (semB0, semB1)

    for ch in range(NCH):
        b = ch & 1
        if ch >= 2:
            pltpu.make_async_copy(
                xbufs[b], xg_hbm.at[p0v.at[ch - 2]], semsA[b]
            ).wait()
            pltpu.make_async_copy(
                xbufs[b], xg_hbm.at[p1v.at[ch - 2]], semsB[b]
            ).wait()
        tok = pl.multiple_of(wid * TW + ch * TCH, 8)
        pltpu.sync_copy(x_hbm.at[pl.ds(tok, TCH)], xbufs[b])
        pltpu.async_copy(xbufs[b], xg_hbm.at[p0v.at[ch]], semsA[b])
        pltpu.async_copy(xbufs[b], xg_hbm.at[p1v.at[ch]], semsB[b])
    for ch in (NCH - 2, NCH - 1):
        b = ch & 1
        pltpu.make_async_copy(xbufs[b], xg_hbm.at[p0v.at[ch]], semsA[b]).wait()
        pltpu.make_async_copy(xbufs[b], xg_hbm.at[p1v.at[ch]], semsB[b]).wait()


def _scatter(x, pos0, pos1):
    mesh = plsc.VectorSubcoreMesh(
        core_axis_name="c", subcore_axis_name="s", num_cores=NC, num_subcores=NS
    )
    f = pl.kernel(
        _scatter_body,
        out_type=jax.ShapeDtypeStruct((P_MAX, D_MODEL // 2), jnp.int32),
        mesh=mesh,
        scratch_types=[
            pltpu.VMEM((NCH, TCH), jnp.int32),
            pltpu.VMEM((NCH, TCH), jnp.int32),
            pltpu.VMEM((TCH, D_MODEL // 2), jnp.int32),
            pltpu.VMEM((TCH, D_MODEL // 2), jnp.int32),
            pltpu.SemaphoreType.DMA,
            pltpu.SemaphoreType.DMA,
            pltpu.SemaphoreType.DMA,
            pltpu.SemaphoreType.DMA,
        ],
        compiler_params=pltpu.CompilerParams(needs_layout_passes=False),
    )
    return f(x, pos0, pos1)


def _gmm_body(be_sref, xg_ref, w_ref, b_ref, y_ref):
    y_ref[...] = (
        jnp.dot(
            xg_ref[...],
            w_ref[0].astype(jnp.bfloat16),
            preferred_element_type=jnp.float32,
        )
        + b_ref[0]
    )


def _grouped_mm(blk, xg, W_experts, b_experts):
    grid_spec = pltpu.PrefetchScalarGridSpec(
        num_scalar_prefetch=1,
        grid=(NB,),
        in_specs=[
            pl.BlockSpec((BB, D_MODEL), lambda i, be: (i, 0)),
            pl.BlockSpec((1, D_MODEL, D_MODEL), lambda i, be: (be[i], 0, 0)),
            pl.BlockSpec((1, 1, D_MODEL), lambda i, be: (be[i], 0, 0)),
        ],
        out_specs=pl.BlockSpec((BB, D_MODEL), lambda i, be: (i, 0)),
    )
    return pl.pallas_call(
        _gmm_body,
        grid_spec=grid_spec,
        out_shape=jax.ShapeDtypeStruct((P_MAX, D_MODEL), jnp.float32),
        compiler_params=pltpu.CompilerParams(
            dimension_semantics=("arbitrary",),
        ),
    )(
        blk,
        xg,
        W_experts,
        b_experts.reshape(NUM_EXPERTS, 1, D_MODEL),
    )


def _unsort_body(
    y_hbm, pos0_hbm, pos1_hbm, g0_hbm, g1_hbm, out_hbm,
    p0v, p1v, g0v, g1v, ya0, ya1, yb0, yb1, ob0, ob1,
    semA0, semB0, semA1, semB1,
):
    c = lax.axis_index("c")
    s = lax.axis_index("s")
    wid = s * NC + c
    row0 = pl.multiple_of(wid * NCH, 8)
    tok0 = pl.multiple_of(wid * TW, 8)
    pltpu.sync_copy(pos0_hbm.at[pl.ds(row0, NCH)], p0v)
    pltpu.sync_copy(pos1_hbm.at[pl.ds(row0, NCH)], p1v)
    pltpu.sync_copy(g0_hbm.at[pl.ds(row0, NCH)], g0v)
    pltpu.sync_copy(g1_hbm.at[pl.ds(row0, NCH)], g1v)

    yas = (ya0, ya1)
    ybs = (yb0, yb1)
    obs = (ob0, ob1)
    semsA = (semA0, semA1)
    semsB = (semB0, semB1)

    def _fire(ch, b):
        pltpu.async_copy(y_hbm.at[p0v.at[ch]], yas[b], semsA[b])
        pltpu.async_copy(y_hbm.at[p1v.at[ch]], ybs[b], semsB[b])

    def _wait(b):
        pltpu.make_async_copy(y_hbm.at[pl.ds(0, TCH)], yas[b], semsA[b]).wait()
        pltpu.make_async_copy(y_hbm.at[pl.ds(0, TCH)], ybs[b], semsB[b]).wait()

    def _combine(ch, b):
        ga = g0v[ch]
        gb = g1v[ch]

        def dbody(d, carry):
            for i in range(TCH):
                obs[b][i, pl.ds(d * L, L)] = (
                    yas[b][i, pl.ds(d * L, L)] * ga[i]
                    + ybs[b][i, pl.ds(d * L, L)] * gb[i]
                )
            return carry

        lax.fori_loop(0, D_MODEL // L, dbody, 0)
        pltpu.sync_copy(
            obs[b], out_hbm.at[pl.ds(pl.multiple_of(tok0 + ch * TCH, 8), TCH)]
        )

    _fire(0, 0)
    for ch in range(NCH):
        b = ch & 1
        if ch + 1 < NCH:
            _fire(ch + 1, 1 - b)
        _wait(b)
        _combine(ch, b)


def _unsort(y, pos0, pos1, g0, g1):
    mesh = plsc.VectorSubcoreMesh(
        core_axis_name="c", subcore_axis_name="s", num_cores=NC, num_subcores=NS
    )
    f = pl.kernel(
        _unsort_body,
        out_type=jax.ShapeDtypeStruct((N_TOKENS, D_MODEL), jnp.float32),
        mesh=mesh,
        scratch_types=[
            pltpu.VMEM((NCH, TCH), jnp.int32),
            pltpu.VMEM((NCH, TCH), jnp.int32),
            pltpu.VMEM((NCH, TCH), jnp.float32),
            pltpu.VMEM((NCH, TCH), jnp.float32),
            pltpu.VMEM((TCH, D_MODEL), jnp.float32),
            pltpu.VMEM((TCH, D_MODEL), jnp.float32),
            pltpu.VMEM((TCH, D_MODEL), jnp.float32),
            pltpu.VMEM((TCH, D_MODEL), jnp.float32),
            pltpu.VMEM((TCH, D_MODEL), jnp.float32),
            pltpu.VMEM((TCH, D_MODEL), jnp.float32),
            pltpu.SemaphoreType.DMA,
            pltpu.SemaphoreType.DMA,
            pltpu.SemaphoreType.DMA,
            pltpu.SemaphoreType.DMA,
        ],
        compiler_params=pltpu.CompilerParams(needs_layout_passes=False),
    )
    return f(y, pos0, pos1, g0, g1)


def kernel(x, W_router, b_router, W_experts, b_experts):
    ei, g0r, g1r, hcnt, xb = _router(x, W_router, b_router)
    pos0, pos1, blk = _plan(ei, hcnt)
    xb32 = jax.lax.bitcast_convert_type(
        xb.reshape(N_TOKENS, D_MODEL // 2, 2), jnp.int32
    )
    xg32 = _scatter(xb32, pos0, pos1)
    xg = jax.lax.bitcast_convert_type(xg32, jnp.bfloat16).reshape(P_MAX, D_MODEL)
    y = _grouped_mm(blk.reshape(NBLK_PAD), xg, W_experts, b_experts)
    return _unsort(y, pos0, pos1, g0r, g1r)


# speedup vs baseline: 2.9603x; 2.9603x over previous
import functools

import jax
import jax.numpy as jnp
from jax import lax
from jax.experimental import pallas as pl
from jax.experimental.pallas import tpu as pltpu
from jax.experimental.pallas import tpu_sc as plsc

NUM_EXPERTS = 16
TOP_K = 2
N_TOKENS = 4096
D_MODEL = 1024
N_PAIRS = N_TOKENS * TOP_K
BB = 256
P_MAX = N_PAIRS + NUM_EXPERTS * BB
NB = P_MAX // BB
NBLK_PAD = 64
NC, NS, L = 2, 16, 16
BT = 1024
NBT = N_TOKENS // BT
TW = N_TOKENS // (NC * NS)
TCH = 16
NCH = TW // TCH


def _router_body(x_ref, wr_ref, br_ref, ei_ref, g0_ref, g1_ref, hcnt_ref):
    logits = (
        jnp.dot(x_ref[...], wr_ref[...], preferred_element_type=jnp.float32)
        + br_ref[...]
    )
    iota = jax.lax.broadcasted_iota(jnp.int32, logits.shape, 1)
    m1 = jnp.max(logits, axis=-1, keepdims=True)
    e0 = jnp.min(
        jnp.where(logits >= m1, iota, NUM_EXPERTS), axis=-1, keepdims=True
    )
    masked = jnp.where(logits >= m1, -jnp.inf, logits)
    m2 = jnp.max(masked, axis=-1, keepdims=True)
    e1 = jnp.min(
        jnp.where(masked >= m2, iota, NUM_EXPERTS), axis=-1, keepdims=True
    )
    t = jnp.exp(m2 - m1)
    g0 = 1.0 / (1.0 + t)
    ei_ref[...] = jnp.concatenate([e0, e1], axis=1)
    g0_ref[...] = g0.reshape(BT // L, L)
    g1_ref[...] = (1.0 - g0).reshape(BT // L, L)
    h = (iota == e0).astype(jnp.float32) + (iota == e1).astype(jnp.float32)
    hcnt_ref[...] = jnp.sum(h, axis=0, keepdims=True).reshape(1, 1, NUM_EXPERTS)


def _router(x, W_router, b_router):
    return pl.pallas_call(
        _router_body,
        grid=(NBT,),
        in_specs=[
            pl.BlockSpec((BT, D_MODEL), lambda i: (i, 0)),
            pl.BlockSpec((D_MODEL, NUM_EXPERTS), lambda i: (0, 0)),
            pl.BlockSpec((1, NUM_EXPERTS), lambda i: (0, 0)),
        ],
        out_specs=[
            pl.BlockSpec((BT, TOP_K), lambda i: (i, 0)),
            pl.BlockSpec((BT // L, L), lambda i: (i, 0)),
            pl.BlockSpec((BT // L, L), lambda i: (i, 0)),
            pl.BlockSpec((1, 1, NUM_EXPERTS), lambda i: (i, 0, 0)),
        ],
        out_shape=[
            jax.ShapeDtypeStruct((N_TOKENS, TOP_K), jnp.int32),
            jax.ShapeDtypeStruct((N_TOKENS // L, L), jnp.float32),
            jax.ShapeDtypeStruct((N_TOKENS // L, L), jnp.float32),
            jax.ShapeDtypeStruct((NBT, 1, NUM_EXPERTS), jnp.float32),
        ],
    )(x, W_router, b_router.reshape(1, NUM_EXPERTS))


def _plan_body(ei_ref, hcnt_ref, pos0_ref, pos1_ref, blk_ref, carry_scr):
    i = pl.program_id(0)

    @pl.when(i == 0)
    def _():
        carry_scr[...] = jnp.zeros_like(carry_scr)

    e0 = ei_ref[:, 0:1]
    e1 = ei_ref[:, 1:2]
    io = jax.lax.broadcasted_iota(jnp.int32, (BT, NUM_EXPERTS), 1)
    oh0 = (io == e0).astype(jnp.float32)
    oh1 = (io == e1).astype(jnp.float32)
    h = oh0 + oh1

    counts = jnp.sum(hcnt_ref[...], axis=0)
    pe = jnp.ceil(counts * (1.0 / BB)) * BB
    r16 = jax.lax.broadcasted_iota(jnp.int32, (NUM_EXPERTS, NUM_EXPERTS), 0)
    c16 = jax.lax.broadcasted_iota(jnp.int32, (NUM_EXPERTS, NUM_EXPERTS), 1)
    tri16 = (r16 <= c16).astype(jnp.float32)
    ends = jnp.dot(pe, tri16, preferred_element_type=jnp.float32)
    base = ends - pe

    rr = jax.lax.broadcasted_iota(jnp.int32, (BT, BT), 0)
    cc = jax.lax.broadcasted_iota(jnp.int32, (BT, BT), 1)
    tstrict = (cc < rr).astype(jnp.float32)
    ex = jnp.dot(tstrict, h, preferred_element_type=jnp.float32) + carry_scr[...]

    rank0 = jnp.sum(ex * oh0, axis=1, keepdims=True)
    rank1 = jnp.sum(ex * oh1, axis=1, keepdims=True)
    base0 = jnp.sum(base * oh0, axis=1, keepdims=True)
    base1 = jnp.sum(base * oh1, axis=1, keepdims=True)
    p0 = (base0 + rank0).astype(jnp.int32)
    p1 = (base1 + rank1).astype(jnp.int32)
    pos0_ref[...] = p0.reshape(BT // L, L)
    pos1_ref[...] = p1.reshape(BT // L, L)

    carry_scr[...] += jnp.sum(h, axis=0, keepdims=True)

    @pl.when(i == NBT - 1)
    def _():
        bstart = (
            jax.lax.broadcasted_iota(jnp.int32, (NBLK_PAD, NUM_EXPERTS), 0) * BB
        ).astype(jnp.float32)
        acc = jnp.sum((bstart >= ends).astype(jnp.float32), axis=1, keepdims=True)
        blk_ref[...] = jnp.minimum(acc, NUM_EXPERTS - 1).astype(jnp.int32)


def _plan(ei, hcnt):
    return pl.pallas_call(
        _plan_body,
        grid=(NBT,),
        in_specs=[
            pl.BlockSpec((BT, TOP_K), lambda i: (i, 0)),
            pl.BlockSpec((NBT, 1, NUM_EXPERTS), lambda i: (0, 0, 0)),
        ],
        out_specs=[
            pl.BlockSpec((BT // L, L), lambda i: (i, 0)),
            pl.BlockSpec((BT // L, L), lambda i: (i, 0)),
            pl.BlockSpec((NBLK_PAD, 1), lambda i: (0, 0)),
        ],
        out_shape=[
            jax.ShapeDtypeStruct((N_TOKENS // L, L), jnp.int32),
            jax.ShapeDtypeStruct((N_TOKENS // L, L), jnp.int32),
            jax.ShapeDtypeStruct((NBLK_PAD, 1), jnp.int32),
        ],
        scratch_shapes=[pltpu.VMEM((1, NUM_EXPERTS), jnp.float32)],
        compiler_params=pltpu.CompilerParams(
            dimension_semantics=("arbitrary",),
        ),
    )(ei, hcnt)


def _scatter_body(
    x_hbm, pos0_hbm, pos1_hbm, xg_hbm,
    p0v, p1v, xb0, xb1, semA0, semB0, semA1, semB1,
):
    c = lax.axis_index("c")
    s = lax.axis_index("s")
    wid = s * NC + c
    row0 = pl.multiple_of(wid * (TW // TCH), 8)
    pltpu.sync_copy(pos0_hbm.at[pl.ds(row0, NCH)], p0v)
    pltpu.sync_copy(pos1_hbm.at[pl.ds(row0, NCH)], p1v)

    xbufs = (xb0, xb1)
    semsA = (semA0, semA1)
    semsB = (semB0, semB1)

    for ch in range(NCH):
        b = ch & 1
        if ch >= 2:
            pltpu.make_async_copy(
                xbufs[b], xg_hbm.at[p0v.at[ch - 2]], semsA[b]
            ).wait()
            pltpu.make_async_copy(
                xbufs[b], xg_hbm.at[p1v.at[ch - 2]], semsB[b]
            ).wait()
        tok = pl.multiple_of(wid * TW + ch * TCH, 8)
        pltpu.sync_copy(x_hbm.at[pl.ds(tok, TCH)], xbufs[b])
        pltpu.async_copy(xbufs[b], xg_hbm.at[p0v.at[ch]], semsA[b])
        pltpu.async_copy(xbufs[b], xg_hbm.at[p1v.at[ch]], semsB[b])
    for ch in (NCH - 2, NCH - 1):
        b = ch & 1
        pltpu.make_async_copy(xbufs[b], xg_hbm.at[p0v.at[ch]], semsA[b]).wait()
        pltpu.make_async_copy(xbufs[b], xg_hbm.at[p1v.at[ch]], semsB[b]).wait()


def _scatter(x, pos0, pos1):
    mesh = plsc.VectorSubcoreMesh(
        core_axis_name="c", subcore_axis_name="s", num_cores=NC, num_subcores=NS
    )
    f = pl.kernel(
        _scatter_body,
        out_type=jax.ShapeDtypeStruct((P_MAX, D_MODEL), jnp.float32),
        mesh=mesh,
        scratch_types=[
            pltpu.VMEM((NCH, TCH), jnp.int32),
            pltpu.VMEM((NCH, TCH), jnp.int32),
            pltpu.VMEM((TCH, D_MODEL), jnp.float32),
            pltpu.VMEM((TCH, D_MODEL), jnp.float32),
            pltpu.SemaphoreType.DMA,
            pltpu.SemaphoreType.DMA,
            pltpu.SemaphoreType.DMA,
            pltpu.SemaphoreType.DMA,
        ],
        compiler_params=pltpu.CompilerParams(needs_layout_passes=False),
    )
    return f(x, pos0, pos1)


def _gmm_body(be_sref, xg_ref, w_ref, b_ref, y_ref):
    y_ref[...] = (
        jnp.dot(
            xg_ref[...].astype(jnp.bfloat16),
            w_ref[0].astype(jnp.bfloat16),
            preferred_element_type=jnp.float32,
        )
        + b_ref[0]
    )


def _grouped_mm(blk, xg, W_experts, b_experts):
    grid_spec = pltpu.PrefetchScalarGridSpec(
        num_scalar_prefetch=1,
        grid=(NB,),
        in_specs=[
            pl.BlockSpec((BB, D_MODEL), lambda i, be: (i, 0)),
            pl.BlockSpec((1, D_MODEL, D_MODEL), lambda i, be: (be[i], 0, 0)),
            pl.BlockSpec((1, 1, D_MODEL), lambda i, be: (be[i], 0, 0)),
        ],
        out_specs=pl.BlockSpec((BB, D_MODEL), lambda i, be: (i, 0)),
    )
    return pl.pallas_call(
        _gmm_body,
        grid_spec=grid_spec,
        out_shape=jax.ShapeDtypeStruct((P_MAX, D_MODEL), jnp.float32),
        compiler_params=pltpu.CompilerParams(
            dimension_semantics=("arbitrary",),
        ),
    )(
        blk,
        xg,
        W_experts,
        b_experts.reshape(NUM_EXPERTS, 1, D_MODEL),
    )


def _unsort_body(
    y_hbm, pos0_hbm, pos1_hbm, g0_hbm, g1_hbm, out_hbm,
    p0v, p1v, g0v, g1v, ya0, ya1, yb0, yb1, ob0, ob1,
    semA0, semB0, semA1, semB1,
):
    c = lax.axis_index("c")
    s = lax.axis_index("s")
    wid = s * NC + c
    row0 = pl.multiple_of(wid * NCH, 8)
    tok0 = pl.multiple_of(wid * TW, 8)
    pltpu.sync_copy(pos0_hbm.at[pl.ds(row0, NCH)], p0v)
    pltpu.sync_copy(pos1_hbm.at[pl.ds(row0, NCH)], p1v)
    pltpu.sync_copy(g0_hbm.at[pl.ds(row0, NCH)], g0v)
    pltpu.sync_copy(g1_hbm.at[pl.ds(row0, NCH)], g1v)

    yas = (ya0, ya1)
    ybs = (yb0, yb1)
    obs = (ob0, ob1)
    semsA = (semA0, semA1)
    semsB = (semB0, semB1)

    def _fire(ch, b):
        pltpu.async_copy(y_hbm.at[p0v.at[ch]], yas[b], semsA[b])
        pltpu.async_copy(y_hbm.at[p1v.at[ch]], ybs[b], semsB[b])

    def _wait(b):
        pltpu.make_async_copy(y_hbm.at[pl.ds(0, TCH)], yas[b], semsA[b]).wait()
        pltpu.make_async_copy(y_hbm.at[pl.ds(0, TCH)], ybs[b], semsB[b]).wait()

    def _combine(ch, b):
        ga = g0v[ch]
        gb = g1v[ch]

        def dbody(d, carry):
            for i in range(TCH):
                obs[b][i, pl.ds(d * L, L)] = (
                    yas[b][i, pl.ds(d * L, L)] * ga[i]
                    + ybs[b][i, pl.ds(d * L, L)] * gb[i]
                )
            return carry

        lax.fori_loop(0, D_MODEL // L, dbody, 0)
        pltpu.sync_copy(
            obs[b], out_hbm.at[pl.ds(pl.multiple_of(tok0 + ch * TCH, 8), TCH)]
        )

    _fire(0, 0)
    for ch in range(NCH):
        b = ch & 1
        if ch + 1 < NCH:
            _fire(ch + 1, 1 - b)
        _wait(b)
        _combine(ch, b)


def _unsort(y, pos0, pos1, g0, g1):
    mesh = plsc.VectorSubcoreMesh(
        core_axis_name="c", subcore_axis_name="s", num_cores=NC, num_subcores=NS
    )
    f = pl.kernel(
        _unsort_body,
        out_type=jax.ShapeDtypeStruct((N_TOKENS, D_MODEL), jnp.float32),
        mesh=mesh,
        scratch_types=[
            pltpu.VMEM((NCH, TCH), jnp.int32),
            pltpu.VMEM((NCH, TCH), jnp.int32),
            pltpu.VMEM((NCH, TCH), jnp.float32),
            pltpu.VMEM((NCH, TCH), jnp.float32),
            pltpu.VMEM((TCH, D_MODEL), jnp.float32),
            pltpu.VMEM((TCH, D_MODEL), jnp.float32),
            pltpu.VMEM((TCH, D_MODEL), jnp.float32),
            pltpu.VMEM((TCH, D_MODEL), jnp.float32),
            pltpu.VMEM((TCH, D_MODEL), jnp.float32),
            pltpu.VMEM((TCH, D_MODEL), jnp.float32),
            pltpu.SemaphoreType.DMA,
            pltpu.SemaphoreType.DMA,
            pltpu.SemaphoreType.DMA,
            pltpu.SemaphoreType.DMA,
        ],
        compiler_params=pltpu.CompilerParams(needs_layout_passes=False),
    )
    return f(y, pos0, pos1, g0, g1)


def kernel(x, W_router, b_router, W_experts, b_experts):
    ei, g0r, g1r, hcnt = _router(x, W_router, b_router)
    pos0, pos1, blk = _plan(ei, hcnt)
    xg = _scatter(x, pos0, pos1)
    y = _grouped_mm(blk.reshape(NBLK_PAD), xg, W_experts, b_experts)
    return _unsort(y, pos0, pos1, g0r, g1r)
